# Initial kernel scaffold; baseline (speedup 1.0000x reference)
import functools

import jax
import jax.numpy as jnp
from jax import lax
from jax.experimental import pallas as pl
from jax.experimental.pallas import tpu as pltpu
from jax.experimental.pallas import tpu_sc as plsc

N = 10000
E = 320000
D = 128

NC = 2
NS = 16
L = 16

CH = 128
NCHUNK = E // CH
ROWS_PER_TILE = N // NS
ZCH = 125


def _make_seg_sum(with_count: bool):
    mesh = plsc.VectorSubcoreMesh(
        core_axis_name="c", subcore_axis_name="s", num_cores=NC,
        num_subcores=NS)

    out_type = [jax.ShapeDtypeStruct((NC, N, D), jnp.float32)]
    scratch = [
        pltpu.VMEM((CH,), jnp.int32),
        pltpu.VMEM((CH,), jnp.int32),
        pltpu.VMEM((CH, D), jnp.float32),
        pltpu.VMEM((ZCH, D), jnp.float32),
        pltpu.VMEM_SHARED((N, D), jnp.float32),
        pltpu.SemaphoreType.DMA,
    ]
    if with_count:
        out_type.append(jax.ShapeDtypeStruct((NC, N, L), jnp.float32))
        scratch += [
            pltpu.VMEM((CH, L), jnp.float32),
            pltpu.VMEM((ZCH, L), jnp.float32),
            pltpu.VMEM_SHARED((N, L), jnp.float32),
        ]

    def body(table_hbm, src_hbm, dst_hbm, out_hbm, *rest):
        if with_count:
            (cnt_hbm, src_v, dst_v, rows_v, zrow_v, acc_sh, sem,
             ones_v, zcnt_v, cnt_sh) = rest
        else:
            src_v, dst_v, rows_v, zrow_v, acc_sh, sem = rest
        cid = lax.axis_index("c")
        sid = lax.axis_index("s")
        wid = sid * NC + cid

        zero16 = jnp.zeros((L,), jnp.float32)

        def zero_row(i, carry):
            for l in range(D // L):
                zrow_v[i, pl.ds(l * L, L)] = zero16
            if with_count:
                @pl.when(i < CH)
                def _():
                    ones_v[i] = jnp.full((L,), 1.0, jnp.float32)
                zcnt_v[i] = zero16
            return carry

        lax.fori_loop(0, ZCH, zero_row, 0)
        if with_count:
            def one_row(i, carry):
                ones_v[i] = jnp.full((L,), 1.0, jnp.float32)
                return carry
            lax.fori_loop(0, CH, one_row, 0)

        for k in range(ROWS_PER_TILE // ZCH):
            r0 = sid * ROWS_PER_TILE + k * ZCH
            pltpu.sync_copy(zrow_v, acc_sh.at[pl.ds(r0, ZCH)])
            if with_count:
                pltpu.sync_copy(zcnt_v, cnt_sh.at[pl.ds(r0, ZCH)])
        plsc.subcore_barrier()

        nj = (NCHUNK + NC * NS - 1) // (NC * NS)

        def chunk(j, carry):
            c = j * (NC * NS) + wid

            @pl.when(c < NCHUNK)
            def _():
                off = c * CH
                pltpu.sync_copy(src_hbm.at[pl.ds(off, CH)], src_v)
                pltpu.sync_copy(dst_hbm.at[pl.ds(off, CH)], dst_v)
                pltpu.async_copy(table_hbm.at[src_v], rows_v, sem).wait()
                pltpu.sync_copy(rows_v, acc_sh.at[dst_v], add=True)
                if with_count:
                    pltpu.sync_copy(ones_v, cnt_sh.at[dst_v], add=True)
            return carry

        lax.fori_loop(0, nj, chunk, 0)
        plsc.subcore_barrier()

        for k in range(ROWS_PER_TILE // ZCH):
            r0 = sid * ROWS_PER_TILE + k * ZCH
            pltpu.sync_copy(acc_sh.at[pl.ds(r0, ZCH)],
                            out_hbm.at[cid, pl.ds(r0, ZCH)])
            if with_count:
                pltpu.sync_copy(cnt_sh.at[pl.ds(r0, ZCH)],
                                cnt_hbm.at[cid, pl.ds(r0, ZCH)])

    return pl.kernel(body, out_type=tuple(out_type), mesh=mesh,
                     scratch_types=tuple(scratch))


_seg_sum_cnt = _make_seg_sum(True)
_seg_sum = _make_seg_sum(False)


_RB = 1000


def _tc1_body(s_ref, c_ref, x_ref, w1l_ref, b1_ref, w1r_ref, w2l_ref,
              h_ref, p_ref):
    s = s_ref[0] + s_ref[1]
    cnt = c_ref[0][:, 0:1] + c_ref[1][:, 0:1]
    agg = s / jnp.maximum(cnt, 1.0)
    hp = lax.dot_general(agg, w1l_ref[...], (((1,), (1,)), ((), ())),
                         preferred_element_type=jnp.float32,
                         precision=lax.Precision.HIGHEST)
    hr = lax.dot_general(x_ref[...], w1r_ref[...], (((1,), (1,)), ((), ())),
                         preferred_element_type=jnp.float32,
                         precision=lax.Precision.HIGHEST)
    h = jnp.maximum(hp + b1_ref[...] + hr, 0.0)
    h_ref[...] = h
    p_ref[...] = lax.dot_general(h, w2l_ref[...], (((1,), (1,)), ((), ())),
                                 preferred_element_type=jnp.float32,
                                 precision=lax.Precision.HIGHEST)


def _tc2_body(s_ref, c_ref, h_ref, w2r_ref, b2_ref, o_ref):
    s = s_ref[0] + s_ref[1]
    cnt = c_ref[0][:, 0:1] + c_ref[1][:, 0:1]
    agg = s / jnp.maximum(cnt, 1.0)
    hr = lax.dot_general(h_ref[...], w2r_ref[...], (((1,), (1,)), ((), ())),
                         preferred_element_type=jnp.float32,
                         precision=lax.Precision.HIGHEST)
    o_ref[...] = agg + b2_ref[...] + hr


def kernel(x, edge_index, W1l, b1, W1r, W2l, b2, W2r):
    H = W1l.shape[0]
    O = W2l.shape[0]
    src = edge_index[0]
    dst = edge_index[1]

    s1, cnt = _seg_sum_cnt(x, src, dst)

    grid = (N // _RB,)
    h, p = pl.pallas_call(
        _tc1_body,
        grid=grid,
        in_specs=[
            pl.BlockSpec((NC, _RB, D), lambda i: (0, i, 0)),
            pl.BlockSpec((NC, _RB, L), lambda i: (0, i, 0)),
            pl.BlockSpec((_RB, D), lambda i: (i, 0)),
            pl.BlockSpec((H, D), lambda i: (0, 0)),
            pl.BlockSpec((1, H), lambda i: (0, 0)),
            pl.BlockSpec((H, D), lambda i: (0, 0)),
            pl.BlockSpec((O, H), lambda i: (0, 0)),
        ],
        out_specs=[
            pl.BlockSpec((_RB, H), lambda i: (i, 0)),
            pl.BlockSpec((_RB, O), lambda i: (i, 0)),
        ],
        out_shape=[
            jax.ShapeDtypeStruct((N, H), jnp.float32),
            jax.ShapeDtypeStruct((N, O), jnp.float32),
        ],
    )(s1, cnt, x, W1l, b1.reshape(1, H), W1r, W2l)

    (s2,) = _seg_sum(p, src, dst)

    out = pl.pallas_call(
        _tc2_body,
        grid=grid,
        in_specs=[
            pl.BlockSpec((NC, _RB, O), lambda i: (0, i, 0)),
            pl.BlockSpec((NC, _RB, L), lambda i: (0, i, 0)),
            pl.BlockSpec((_RB, H), lambda i: (i, 0)),
            pl.BlockSpec((O, H), lambda i: (0, 0)),
            pl.BlockSpec((1, O), lambda i: (0, 0)),
        ],
        out_specs=pl.BlockSpec((_RB, O), lambda i: (i, 0)),
        out_shape=jax.ShapeDtypeStruct((N, O), jnp.float32),
    )(s2, cnt, h, W2r, b2.reshape(1, O))
    return out


# speedup vs baseline: 6.0695x; 6.0695x over previous
import functools

import jax
import jax.numpy as jnp
from jax import lax
from jax.experimental import pallas as pl
from jax.experimental.pallas import tpu as pltpu
from jax.experimental.pallas import tpu_sc as plsc

N = 10000
E = 320000
D = 128

NC = 2
NS = 16
L = 16

CH = 128
NCHUNK = E // CH
ZCH = 80
NZ = N // ZCH


def _make_seg_sum():
    mesh = plsc.VectorSubcoreMesh(
        core_axis_name="c", subcore_axis_name="s", num_cores=NC,
        num_subcores=NS)

    def body(table_hbm, src_hbm, dst_hbm, out_hbm,
             src_v, dst_v, rows_v, zrow_v, acc_sh, sem):
        cid = lax.axis_index("c")
        sid = lax.axis_index("s")
        wid = sid * NC + cid

        zero16 = jnp.zeros((L,), jnp.float32)

        def zero_row(i, carry):
            for l in range(D // L):
                zrow_v[i, pl.ds(l * L, L)] = zero16
            return carry

        lax.fori_loop(0, ZCH, zero_row, 0)

        def zero_chunk(j, carry):
            m = j * NS + sid

            @pl.when(m < NZ)
            def _():
                pltpu.sync_copy(zrow_v, acc_sh.at[pl.ds(m * ZCH, ZCH)])
            return carry

        lax.fori_loop(0, (NZ + NS - 1) // NS, zero_chunk, 0)
        plsc.subcore_barrier()

        def chunk(j, carry):
            c = j * (NC * NS) + wid

            @pl.when(c < NCHUNK)
            def _():
                off = c * CH
                pltpu.sync_copy(src_hbm.at[pl.ds(off, CH)], src_v)
                pltpu.sync_copy(dst_hbm.at[pl.ds(off, CH)], dst_v)
                pltpu.async_copy(table_hbm.at[src_v], rows_v, sem).wait()
                pltpu.sync_copy(rows_v, acc_sh.at[dst_v], add=True)
            return carry

        lax.fori_loop(0, (NCHUNK + NC * NS - 1) // (NC * NS), chunk, 0)
        plsc.subcore_barrier()

        def write_chunk(j, carry):
            m = j * NS + sid

            @pl.when(m < NZ)
            def _():
                pltpu.sync_copy(acc_sh.at[pl.ds(m * ZCH, ZCH)],
                                out_hbm.at[cid, pl.ds(m * ZCH, ZCH)])
            return carry

        lax.fori_loop(0, (NZ + NS - 1) // NS, write_chunk, 0)

    return pl.kernel(
        body,
        out_type=(jax.ShapeDtypeStruct((NC, N, D), jnp.float32),),
        mesh=mesh,
        scratch_types=(
            pltpu.VMEM((CH,), jnp.int32),
            pltpu.VMEM((CH,), jnp.int32),
            pltpu.VMEM((CH, D), jnp.float32),
            pltpu.VMEM((ZCH, D), jnp.float32),
            pltpu.VMEM_SHARED((N, D), jnp.float32),
            pltpu.SemaphoreType.DMA,
        ))


def _make_count():
    mesh = plsc.VectorSubcoreMesh(
        core_axis_name="c", subcore_axis_name="s", num_cores=NC,
        num_subcores=NS)

    def body(dst_hbm, out_hbm, dst_v, ones_v, zcnt_v, cnt_sh):
        cid = lax.axis_index("c")
        sid = lax.axis_index("s")
        wid = sid * NC + cid

        def fill_row(i, carry):
            for l in range(D // L):
                @pl.when(i < ZCH)
                def _():
                    zcnt_v[i, pl.ds(l * L, L)] = jnp.zeros((L,), jnp.float32)
                ones_v[i, pl.ds(l * L, L)] = jnp.full((L,), 1.0, jnp.float32)
            return carry

        lax.fori_loop(0, CH, fill_row, 0)

        def zero_chunk(j, carry):
            m = j * NS + sid

            @pl.when(m < NZ)
            def _():
                pltpu.sync_copy(zcnt_v, cnt_sh.at[pl.ds(m * ZCH, ZCH)])
            return carry

        lax.fori_loop(0, (NZ + NS - 1) // NS, zero_chunk, 0)
        plsc.subcore_barrier()

        def chunk(j, carry):
            c = j * (NC * NS) + wid

            @pl.when(c < NCHUNK)
            def _():
                pltpu.sync_copy(dst_hbm.at[pl.ds(c * CH, CH)], dst_v)
                pltpu.sync_copy(ones_v, cnt_sh.at[dst_v], add=True)
            return carry

        lax.fori_loop(0, (NCHUNK + NC * NS - 1) // (NC * NS), chunk, 0)
        plsc.subcore_barrier()

        def write_chunk(j, carry):
            m = j * NS + sid

            @pl.when(m < NZ)
            def _():
                pltpu.sync_copy(cnt_sh.at[pl.ds(m * ZCH, ZCH)],
                                out_hbm.at[cid, pl.ds(m * ZCH, ZCH)])
            return carry

        lax.fori_loop(0, (NZ + NS - 1) // NS, write_chunk, 0)

    return pl.kernel(
        body,
        out_type=(jax.ShapeDtypeStruct((NC, N, D), jnp.float32),),
        mesh=mesh,
        scratch_types=(
            pltpu.VMEM((CH,), jnp.int32),
            pltpu.VMEM((CH, D), jnp.float32),
            pltpu.VMEM((ZCH, D), jnp.float32),
            pltpu.VMEM_SHARED((N, D), jnp.float32),
        ))


_seg_sum = _make_seg_sum()
_count = _make_count()


_RB = 1000


def _tc1_body(s_ref, c_ref, x_ref, w1l_ref, b1_ref, w1r_ref, w2l_ref,
              h_ref, p_ref):
    s = s_ref[0] + s_ref[1]
    cnt = c_ref[0][:, 0:1] + c_ref[1][:, 0:1]
    agg = s / jnp.maximum(cnt, 1.0)
    hp = lax.dot_general(agg, w1l_ref[...], (((1,), (1,)), ((), ())),
                         preferred_element_type=jnp.float32,
                         precision=lax.Precision.HIGHEST)
    hr = lax.dot_general(x_ref[...], w1r_ref[...], (((1,), (1,)), ((), ())),
                         preferred_element_type=jnp.float32,
                         precision=lax.Precision.HIGHEST)
    h = jnp.maximum(hp + b1_ref[...] + hr, 0.0)
    h_ref[...] = h
    p_ref[...] = lax.dot_general(h, w2l_ref[...], (((1,), (1,)), ((), ())),
                                 preferred_element_type=jnp.float32,
                                 precision=lax.Precision.HIGHEST)


def _tc2_body(s_ref, c_ref, h_ref, w2r_ref, b2_ref, o_ref):
    s = s_ref[0] + s_ref[1]
    cnt = c_ref[0][:, 0:1] + c_ref[1][:, 0:1]
    agg = s / jnp.maximum(cnt, 1.0)
    hr = lax.dot_general(h_ref[...], w2r_ref[...], (((1,), (1,)), ((), ())),
                         preferred_element_type=jnp.float32,
                         precision=lax.Precision.HIGHEST)
    o_ref[...] = agg + b2_ref[...] + hr


def kernel(x, edge_index, W1l, b1, W1r, W2l, b2, W2r):
    H = W1l.shape[0]
    O = W2l.shape[0]
    src = edge_index[0]
    dst = edge_index[1]

    (cnt,) = _count(dst)
    (s1,) = _seg_sum(x, src, dst)

    grid = (N // _RB,)
    h, p = pl.pallas_call(
        _tc1_body,
        grid=grid,
        in_specs=[
            pl.BlockSpec((NC, _RB, D), lambda i: (0, i, 0)),
            pl.BlockSpec((NC, _RB, D), lambda i: (0, i, 0)),
            pl.BlockSpec((_RB, D), lambda i: (i, 0)),
            pl.BlockSpec((H, D), lambda i: (0, 0)),
            pl.BlockSpec((1, H), lambda i: (0, 0)),
            pl.BlockSpec((H, D), lambda i: (0, 0)),
            pl.BlockSpec((O, H), lambda i: (0, 0)),
        ],
        out_specs=[
            pl.BlockSpec((_RB, H), lambda i: (i, 0)),
            pl.BlockSpec((_RB, O), lambda i: (i, 0)),
        ],
        out_shape=[
            jax.ShapeDtypeStruct((N, H), jnp.float32),
            jax.ShapeDtypeStruct((N, O), jnp.float32),
        ],
    )(s1, cnt, x, W1l, b1.reshape(1, H), W1r, W2l)

    (s2,) = _seg_sum(p, src, dst)

    out = pl.pallas_call(
        _tc2_body,
        grid=grid,
        in_specs=[
            pl.BlockSpec((NC, _RB, O), lambda i: (0, i, 0)),
            pl.BlockSpec((NC, _RB, D), lambda i: (0, i, 0)),
            pl.BlockSpec((_RB, H), lambda i: (i, 0)),
            pl.BlockSpec((O, H), lambda i: (0, 0)),
            pl.BlockSpec((1, O), lambda i: (0, 0)),
        ],
        out_specs=pl.BlockSpec((_RB, O), lambda i: (i, 0)),
        out_shape=jax.ShapeDtypeStruct((N, O), jnp.float32),
    )(s2, cnt, h, W2r, b2.reshape(1, O))
    return out


# speedup vs baseline: 9.5528x; 1.5739x over previous
import functools

import jax
import jax.numpy as jnp
from jax import lax
from jax.experimental import pallas as pl
from jax.experimental.pallas import tpu as pltpu
from jax.experimental.pallas import tpu_sc as plsc

N = 10000
E = 320000
D = 128

NC = 2
NS = 16
L = 16

CW = 125
NCH = E // CW
NCHT = NCH // (NC * NS)
IB = 16
ZCH = 80
NZ = N // ZCH


def _zero_rows(ref, nrows):
    zero16 = jnp.zeros((L,), jnp.float32)

    def row(i, carry):
        for l in range(D // L):
            ref[i, pl.ds(l * L, L)] = zero16
        return carry

    lax.fori_loop(0, nrows, row, 0)


def _round_robin_copy(sid, body_fn):
    def step(j, carry):
        m = j * NS + sid

        @pl.when(m < NZ)
        def _():
            body_fn(m * ZCH)
        return carry

    lax.fori_loop(0, (NZ + NS - 1) // NS, step, 0)


def _make_seg_sum():
    mesh = plsc.VectorSubcoreMesh(
        core_axis_name="c", subcore_axis_name="s", num_cores=NC,
        num_subcores=NS)

    def body(table_hbm, src_hbm, dst_hbm, out_hbm,
             src_v, dst_v, rows_v, zrow_v, acc_sh,
             semg0, semg1, sems0, sems1):
        cid = lax.axis_index("c")
        sid = lax.axis_index("s")
        wid = sid * NC + cid

        _zero_rows(zrow_v, ZCH)
        _round_robin_copy(
            sid, lambda r0: pltpu.sync_copy(zrow_v, acc_sh.at[pl.ds(r0, ZCH)]))
        plsc.subcore_barrier()

        semg = (semg0, semg1)
        sems = (sems0, sems1)

        def start_gather(j, b):
            pltpu.async_copy(table_hbm.at[src_v.at[j]], rows_v.at[b], semg[b])

        def wait_gather(b):
            pltpu.make_async_copy(
                table_hbm.at[src_v.at[0]], rows_v.at[b], semg[b]).wait()

        def start_scatter(j, b):
            pltpu.async_copy(rows_v.at[b], acc_sh.at[dst_v.at[j]], sems[b],
                             add=True)

        def wait_scatter(b):
            pltpu.make_async_copy(
                rows_v.at[b], acc_sh.at[dst_v.at[0]], sems[b]).wait()

        def block(bi, carry):
            c0 = wid * NCHT + bi * IB
            pltpu.sync_copy(src_hbm.at[pl.ds(c0, IB)], src_v)
            pltpu.sync_copy(dst_hbm.at[pl.ds(c0, IB)], dst_v)

            start_gather(0, 0)

            def pair(jj, c):
                for b in range(2):
                    j = jj * 2 + b
                    wait_gather(b)
                    start_scatter(j, b)

                    @pl.when(j >= 1)
                    def _():
                        wait_scatter(1 - b)

                    @pl.when(j + 1 < IB)
                    def _():
                        start_gather(j + 1, 1 - b)
                return c

            lax.fori_loop(0, IB // 2, pair, 0)
            wait_scatter(1)
            return carry

        lax.fori_loop(0, NCHT // IB, block, 0)
        plsc.subcore_barrier()

        _round_robin_copy(
            sid, lambda r0: pltpu.sync_copy(acc_sh.at[pl.ds(r0, ZCH)],
                                            out_hbm.at[cid, pl.ds(r0, ZCH)]))

    return pl.kernel(
        body,
        out_type=(jax.ShapeDtypeStruct((NC, N, D), jnp.float32),),
        mesh=mesh,
        scratch_types=(
            pltpu.VMEM((IB, CW), jnp.int32),
            pltpu.VMEM((IB, CW), jnp.int32),
            pltpu.VMEM((2, CW, D), jnp.float32),
            pltpu.VMEM((ZCH, D), jnp.float32),
            pltpu.VMEM_SHARED((N, D), jnp.float32),
            pltpu.SemaphoreType.DMA,
            pltpu.SemaphoreType.DMA,
            pltpu.SemaphoreType.DMA,
            pltpu.SemaphoreType.DMA,
        ))


def _make_count():
    mesh = plsc.VectorSubcoreMesh(
        core_axis_name="c", subcore_axis_name="s", num_cores=NC,
        num_subcores=NS)

    def body(dst_hbm, out_hbm, dst_v, ones_v, zrow_v, cnt_sh, sem0, sem1):
        cid = lax.axis_index("c")
        sid = lax.axis_index("s")
        wid = sid * NC + cid

        pltpu.sync_copy(dst_hbm.at[pl.ds(wid * NCHT, NCHT)], dst_v)

        one16 = jnp.full((L,), 1.0, jnp.float32)

        def fill_row(i, carry):
            for l in range(D // L):
                @pl.when(i < ZCH)
                def _():
                    zrow_v[i, pl.ds(l * L, L)] = jnp.zeros((L,), jnp.float32)
                ones_v[i, pl.ds(l * L, L)] = one16
            return carry

        lax.fori_loop(0, CW, fill_row, 0)
        _round_robin_copy(
            sid, lambda r0: pltpu.sync_copy(zrow_v, cnt_sh.at[pl.ds(r0, ZCH)]))
        plsc.subcore_barrier()

        sems = (sem0, sem1)

        def pair(jj, carry):
            for b in range(2):
                j = jj * 2 + b

                @pl.when(j >= 2)
                def _():
                    pltpu.make_async_copy(
                        ones_v, cnt_sh.at[dst_v.at[0]], sems[b]).wait()

                pltpu.async_copy(ones_v, cnt_sh.at[dst_v.at[j]], sems[b],
                                 add=True)
            return carry

        lax.fori_loop(0, NCHT // 2, pair, 0)
        for b in range(2):
            pltpu.make_async_copy(
                ones_v, cnt_sh.at[dst_v.at[0]], sems[b]).wait()
        plsc.subcore_barrier()

        _round_robin_copy(
            sid, lambda r0: pltpu.sync_copy(cnt_sh.at[pl.ds(r0, ZCH)],
                                            out_hbm.at[cid, pl.ds(r0, ZCH)]))

    return pl.kernel(
        body,
        out_type=(jax.ShapeDtypeStruct((NC, N, D), jnp.float32),),
        mesh=mesh,
        scratch_types=(
            pltpu.VMEM((NCHT, CW), jnp.int32),
            pltpu.VMEM((CW, D), jnp.float32),
            pltpu.VMEM((ZCH, D), jnp.float32),
            pltpu.VMEM_SHARED((N, D), jnp.float32),
            pltpu.SemaphoreType.DMA,
            pltpu.SemaphoreType.DMA,
        ))


_seg_sum = _make_seg_sum()
_count = _make_count()


_RB = 1000


def _tc1_body(s_ref, c_ref, x_ref, w1l_ref, b1_ref, w1r_ref, w2l_ref,
              h_ref, p_ref):
    s = s_ref[0] + s_ref[1]
    cnt = c_ref[0][:, 0:1] + c_ref[1][:, 0:1]
    agg = s / jnp.maximum(cnt, 1.0)
    hp = lax.dot_general(agg, w1l_ref[...], (((1,), (1,)), ((), ())),
                         preferred_element_type=jnp.float32,
                         precision=lax.Precision.HIGHEST)
    hr = lax.dot_general(x_ref[...], w1r_ref[...], (((1,), (1,)), ((), ())),
                         preferred_element_type=jnp.float32,
                         precision=lax.Precision.HIGHEST)
    h = jnp.maximum(hp + b1_ref[...] + hr, 0.0)
    h_ref[...] = h
    p_ref[...] = lax.dot_general(h, w2l_ref[...], (((1,), (1,)), ((), ())),
                                 preferred_element_type=jnp.float32,
                                 precision=lax.Precision.HIGHEST)


def _tc2_body(s_ref, c_ref, h_ref, w2r_ref, b2_ref, o_ref):
    s = s_ref[0] + s_ref[1]
    cnt = c_ref[0][:, 0:1] + c_ref[1][:, 0:1]
    agg = s / jnp.maximum(cnt, 1.0)
    hr = lax.dot_general(h_ref[...], w2r_ref[...], (((1,), (1,)), ((), ())),
                         preferred_element_type=jnp.float32,
                         precision=lax.Precision.HIGHEST)
    o_ref[...] = agg + b2_ref[...] + hr


def kernel(x, edge_index, W1l, b1, W1r, W2l, b2, W2r):
    H = W1l.shape[0]
    O = W2l.shape[0]
    src = edge_index[0].reshape(NCH, CW)
    dst = edge_index[1].reshape(NCH, CW)

    (cnt,) = _count(dst)
    (s1,) = _seg_sum(x, src, dst)

    grid = (N // _RB,)
    h, p = pl.pallas_call(
        _tc1_body,
        grid=grid,
        in_specs=[
            pl.BlockSpec((NC, _RB, D), lambda i: (0, i, 0)),
            pl.BlockSpec((NC, _RB, D), lambda i: (0, i, 0)),
            pl.BlockSpec((_RB, D), lambda i: (i, 0)),
            pl.BlockSpec((H, D), lambda i: (0, 0)),
            pl.BlockSpec((1, H), lambda i: (0, 0)),
            pl.BlockSpec((H, D), lambda i: (0, 0)),
            pl.BlockSpec((O, H), lambda i: (0, 0)),
        ],
        out_specs=[
            pl.BlockSpec((_RB, H), lambda i: (i, 0)),
            pl.BlockSpec((_RB, O), lambda i: (i, 0)),
        ],
        out_shape=[
            jax.ShapeDtypeStruct((N, H), jnp.float32),
            jax.ShapeDtypeStruct((N, O), jnp.float32),
        ],
    )(s1, cnt, x, W1l, b1.reshape(1, H), W1r, W2l)

    (s2,) = _seg_sum(p, src, dst)

    out = pl.pallas_call(
        _tc2_body,
        grid=grid,
        in_specs=[
            pl.BlockSpec((NC, _RB, O), lambda i: (0, i, 0)),
            pl.BlockSpec((NC, _RB, D), lambda i: (0, i, 0)),
            pl.BlockSpec((_RB, H), lambda i: (i, 0)),
            pl.BlockSpec((O, H), lambda i: (0, 0)),
            pl.BlockSpec((1, O), lambda i: (0, 0)),
        ],
        out_specs=pl.BlockSpec((_RB, O), lambda i: (i, 0)),
        out_shape=jax.ShapeDtypeStruct((N, O), jnp.float32),
    )(s2, cnt, h, W2r, b2.reshape(1, O))
    return out


# speedup vs baseline: 10.4818x; 1.0973x over previous
import functools

import jax
import jax.numpy as jnp
from jax import lax
from jax.experimental import pallas as pl
from jax.experimental.pallas import tpu as pltpu
from jax.experimental.pallas import tpu_sc as plsc

N = 10000
E = 320000
D = 128

NC = 2
NS = 16
L = 16

CW = 125
NCH = E // CW
NCHT = NCH // (NC * NS)
IB = 16
ZCH = 80
NZ = N // ZCH
CL = 8


def _zero_rows(ref, nrows, width=D):
    zero16 = jnp.zeros((L,), jnp.float32)

    def row(i, carry):
        for l in range(width // L):
            ref[i, pl.ds(l * L, L)] = zero16
        return carry

    lax.fori_loop(0, nrows, row, 0)


def _round_robin_copy(sid, body_fn, nz=NZ, zch=ZCH):
    def step(j, carry):
        m = j * NS + sid

        @pl.when(m < nz)
        def _():
            body_fn(m * zch)
        return carry

    lax.fori_loop(0, (nz + NS - 1) // NS, step, 0)


def _make_seg_sum():
    mesh = plsc.VectorSubcoreMesh(
        core_axis_name="c", subcore_axis_name="s", num_cores=NC,
        num_subcores=NS)

    def body(table_hbm, src_hbm, dst_hbm, out_hbm,
             src_v, dst_v, rows_v, zrow_v, acc_sh,
             semg0, semg1, sems0, sems1):
        cid = lax.axis_index("c")
        sid = lax.axis_index("s")
        wid = sid * NC + cid

        _zero_rows(zrow_v, ZCH)
        _round_robin_copy(
            sid, lambda r0: pltpu.sync_copy(zrow_v, acc_sh.at[pl.ds(r0, ZCH)]))
        plsc.subcore_barrier()

        semg = (semg0, semg1)
        sems = (sems0, sems1)

        def start_gather(j, b):
            pltpu.async_copy(table_hbm.at[src_v.at[j]], rows_v.at[b], semg[b])

        def wait_gather(b):
            pltpu.make_async_copy(
                table_hbm.at[src_v.at[0]], rows_v.at[b], semg[b]).wait()

        def start_scatter(j, b):
            pltpu.async_copy(rows_v.at[b], acc_sh.at[dst_v.at[j]], sems[b],
                             add=True)

        def wait_scatter(b):
            pltpu.make_async_copy(
                rows_v.at[b], acc_sh.at[dst_v.at[0]], sems[b]).wait()

        def block(bi, carry):
            c0 = wid * NCHT + bi * IB
            pltpu.sync_copy(src_hbm.at[pl.ds(c0, IB)], src_v)
            pltpu.sync_copy(dst_hbm.at[pl.ds(c0, IB)], dst_v)

            start_gather(0, 0)

            def pair(jj, c):
                for b in range(2):
                    j = jj * 2 + b
                    wait_gather(b)
                    start_scatter(j, b)

                    @pl.when(j >= 1)
                    def _():
                        wait_scatter(1 - b)

                    @pl.when(j + 1 < IB)
                    def _():
                        start_gather(j + 1, 1 - b)
                return c

            lax.fori_loop(0, IB // 2, pair, 0)
            wait_scatter(1)
            return carry

        lax.fori_loop(0, NCHT // IB, block, 0)
        plsc.subcore_barrier()

        _round_robin_copy(
            sid, lambda r0: pltpu.sync_copy(acc_sh.at[pl.ds(r0, ZCH)],
                                            out_hbm.at[cid, pl.ds(r0, ZCH)]))

    return pl.kernel(
        body,
        out_type=(jax.ShapeDtypeStruct((NC, N, D), jnp.float32),),
        mesh=mesh,
        scratch_types=(
            pltpu.VMEM((IB, CW), jnp.int32),
            pltpu.VMEM((IB, CW), jnp.int32),
            pltpu.VMEM((2, CW, D), jnp.float32),
            pltpu.VMEM((ZCH, D), jnp.float32),
            pltpu.VMEM_SHARED((N, D), jnp.float32),
            pltpu.SemaphoreType.DMA,
            pltpu.SemaphoreType.DMA,
            pltpu.SemaphoreType.DMA,
            pltpu.SemaphoreType.DMA,
        ))


def _make_count():
    mesh = plsc.VectorSubcoreMesh(
        core_axis_name="c", subcore_axis_name="s", num_cores=NC,
        num_subcores=NS)

    def body(dst_hbm, ones_hbm, zcnt_hbm, out_hbm,
             dst_v, ones_v, cnt_sh, sem0, sem1):
        cid = lax.axis_index("c")
        sid = lax.axis_index("s")
        wid = sid * NC + cid

        pltpu.sync_copy(dst_hbm.at[pl.ds(wid * NCHT, NCHT)], dst_v)
        pltpu.sync_copy(ones_hbm, ones_v)
        _round_robin_copy(
            sid, lambda r0: pltpu.sync_copy(zcnt_hbm.at[pl.ds(r0, ZCH)],
                                            cnt_sh.at[pl.ds(r0, ZCH)]))
        plsc.subcore_barrier()

        sems = (sem0, sem1)

        def pair(jj, carry):
            for b in range(2):
                j = jj * 2 + b

                @pl.when(j >= 2)
                def _():
                    pltpu.make_async_copy(
                        ones_v, cnt_sh.at[dst_v.at[0]], sems[b]).wait()

                pltpu.async_copy(ones_v, cnt_sh.at[dst_v.at[j]], sems[b],
                                 add=True)
            return carry

        lax.fori_loop(0, NCHT // 2, pair, 0)
        for b in range(2):
            pltpu.make_async_copy(
                ones_v, cnt_sh.at[dst_v.at[0]], sems[b]).wait()
        plsc.subcore_barrier()

        _round_robin_copy(
            sid, lambda r0: pltpu.sync_copy(cnt_sh.at[pl.ds(r0, ZCH)],
                                            out_hbm.at[cid, pl.ds(r0, ZCH)]))

    return pl.kernel(
        body,
        out_type=(jax.ShapeDtypeStruct((NC, N, CL), jnp.float32),),
        mesh=mesh,
        scratch_types=(
            pltpu.VMEM((NCHT, CW), jnp.int32),
            pltpu.VMEM((CW, CL), jnp.float32),
            pltpu.VMEM_SHARED((N, CL), jnp.float32),
            pltpu.SemaphoreType.DMA,
            pltpu.SemaphoreType.DMA,
        ))


_seg_sum = _make_seg_sum()
_count = _make_count()


_RB = 1000


def _tc1_body(s_ref, c_ref, x_ref, w1l_ref, b1_ref, w1r_ref, w2l_ref,
              h_ref, p_ref):
    s = s_ref[0] + s_ref[1]
    cnt = c_ref[0][:, 0:1] + c_ref[1][:, 0:1]
    agg = s / jnp.maximum(cnt, 1.0)
    hp = lax.dot_general(agg, w1l_ref[...], (((1,), (1,)), ((), ())),
                         preferred_element_type=jnp.float32,
                         precision=lax.Precision.HIGHEST)
    hr = lax.dot_general(x_ref[...], w1r_ref[...], (((1,), (1,)), ((), ())),
                         preferred_element_type=jnp.float32,
                         precision=lax.Precision.HIGHEST)
    h = jnp.maximum(hp + b1_ref[...] + hr, 0.0)
    h_ref[...] = h
    p_ref[...] = lax.dot_general(h, w2l_ref[...], (((1,), (1,)), ((), ())),
                                 preferred_element_type=jnp.float32,
                                 precision=lax.Precision.HIGHEST)


def _tc2_body(s_ref, c_ref, h_ref, w2r_ref, b2_ref, o_ref):
    s = s_ref[0] + s_ref[1]
    cnt = c_ref[0][:, 0:1] + c_ref[1][:, 0:1]
    agg = s / jnp.maximum(cnt, 1.0)
    hr = lax.dot_general(h_ref[...], w2r_ref[...], (((1,), (1,)), ((), ())),
                         preferred_element_type=jnp.float32,
                         precision=lax.Precision.HIGHEST)
    o_ref[...] = agg + b2_ref[...] + hr


def kernel(x, edge_index, W1l, b1, W1r, W2l, b2, W2r):
    H = W1l.shape[0]
    O = W2l.shape[0]
    src = edge_index[0].reshape(NCH, CW)
    dst = edge_index[1].reshape(NCH, CW)
    ones = jnp.ones((CW, CL), jnp.float32)
    zcnt = jnp.zeros((N, CL), jnp.float32)

    (cnt,) = _count(dst, ones, zcnt)
    (s1,) = _seg_sum(x, src, dst)

    grid = (N // _RB,)
    h, p = pl.pallas_call(
        _tc1_body,
        grid=grid,
        in_specs=[
            pl.BlockSpec((NC, _RB, D), lambda i: (0, i, 0)),
            pl.BlockSpec((NC, _RB, CL), lambda i: (0, i, 0)),
            pl.BlockSpec((_RB, D), lambda i: (i, 0)),
            pl.BlockSpec((H, D), lambda i: (0, 0)),
            pl.BlockSpec((1, H), lambda i: (0, 0)),
            pl.BlockSpec((H, D), lambda i: (0, 0)),
            pl.BlockSpec((O, H), lambda i: (0, 0)),
        ],
        out_specs=[
            pl.BlockSpec((_RB, H), lambda i: (i, 0)),
            pl.BlockSpec((_RB, O), lambda i: (i, 0)),
        ],
        out_shape=[
            jax.ShapeDtypeStruct((N, H), jnp.float32),
            jax.ShapeDtypeStruct((N, O), jnp.float32),
        ],
    )(s1, cnt, x, W1l, b1.reshape(1, H), W1r, W2l)

    (s2,) = _seg_sum(p, src, dst)

    out = pl.pallas_call(
        _tc2_body,
        grid=grid,
        in_specs=[
            pl.BlockSpec((NC, _RB, O), lambda i: (0, i, 0)),
            pl.BlockSpec((NC, _RB, CL), lambda i: (0, i, 0)),
            pl.BlockSpec((_RB, H), lambda i: (i, 0)),
            pl.BlockSpec((O, H), lambda i: (0, 0)),
            pl.BlockSpec((1, O), lambda i: (0, 0)),
        ],
        out_specs=pl.BlockSpec((_RB, O), lambda i: (i, 0)),
        out_shape=jax.ShapeDtypeStruct((N, O), jnp.float32),
    )(s2, cnt, h, W2r, b2.reshape(1, O))
    return out
